# Initial kernel scaffold; baseline (speedup 1.0000x reference)
import jax
import jax.numpy as jnp
from jax.experimental import pallas as pl


def kernel(x):
    raise NotImplementedError("write your pallas kernel here")


# speedup vs baseline: 23.9156x; 23.9156x over previous
import jax
import jax.numpy as jnp
from jax.experimental import pallas as pl
from jax.experimental.pallas import tpu as pltpu

_B = 2
_N = 2048
_R = 256
_NB = _N // _R


def _body(x_ref, o_ref, carry_ref):
    j = pl.program_id(1)

    @pl.when(j == 0)
    def _():
        carry_ref[...] = jnp.zeros_like(carry_ref)

    x = x_ref[...]
    row = jax.lax.broadcasted_iota(jnp.int32, (_R, _R), 0)
    col = jax.lax.broadcasted_iota(jnp.int32, (_R, _R), 1)
    u = (col >= row).astype(jnp.float32)
    s = jnp.dot(u, x, preferred_element_type=jnp.float32)
    c = carry_ref[...]
    o_ref[...] = s + c
    carry_ref[...] = c + s[0:1, :]


def kernel(x):
    return pl.pallas_call(
        _body,
        grid=(_B, _NB),
        in_specs=[pl.BlockSpec((None, _R, _N), lambda b, j: (b, _NB - 1 - j, 0))],
        out_specs=pl.BlockSpec((None, _R, _N), lambda b, j: (b, _NB - 1 - j, 0)),
        out_shape=jax.ShapeDtypeStruct((_B, _N, _N), jnp.float32),
        scratch_shapes=[pltpu.VMEM((1, _N), jnp.float32)],
    )(x)
